# Initial kernel scaffold; baseline (speedup 1.0000x reference)
import jax
import jax.numpy as jnp
from jax.experimental import pallas as pl


def kernel(x, duty_cycle):
    raise NotImplementedError("write your pallas kernel here")


# speedup vs baseline: 2.0125x; 2.0125x over previous
import functools

import numpy as np
import jax
import jax.numpy as jnp
from jax import lax
from jax.experimental import pallas as pl
from jax.experimental.pallas import tpu as pltpu
from jax.experimental.pallas import tpu_sc as plsc

N = 8192
B = 64
K = 164
TD = float(K) / float(N)
L = 16
NV = N // L
NC = 2
NS = 16
NW = NC * NS
ROWS_PER_W = B // NW

_SIGN = np.int32(-0x80000000)
_MANT = np.int32(0x7FFFFFFF)


def _tile_body(x_hbm, dc_hbm, out_hbm, xrow, boost, keys):
    wid = lax.axis_index("s") * NC + lax.axis_index("c")
    base = wid * ROWS_PER_W
    pltpu.sync_copy(x_hbm.at[pl.ds(base, ROWS_PER_W)], xrow)
    pltpu.sync_copy(dc_hbm, boost)

    def boost_body(i, _):
        d = boost[pl.ds(i * L, L)]
        boost[pl.ds(i * L, L)] = jnp.exp(TD - d)
        return 0

    lax.fori_loop(0, NV, boost_body, 0)

    for r in range(ROWS_PER_W):
        def keys_body(i, _):
            xv = xrow[r, pl.ds(i * L, L)]
            p = xv * boost[pl.ds(i * L, L)]
            kb = lax.bitcast_convert_type(p, jnp.int32)
            keys[pl.ds(i * L, L)] = jnp.where(kb < 0, kb ^ _MANT, kb)
            return 0

        lax.fori_loop(0, NV, keys_body, 0)

        def bit_body(bi, up):
            tbit = lax.shift_left(jnp.int32(1), jnp.int32(31) - bi)
            cand = up | tbit
            t_signed = cand ^ _SIGN

            def cnt_body(i, acc):
                kv = keys[pl.ds(i * L, L)]
                return acc + jnp.where(kv >= t_signed, 1, 0).astype(jnp.int32)

            acc = lax.fori_loop(
                0, NV, cnt_body, jnp.zeros((L,), jnp.int32)
            )
            cnt = jnp.sum(acc)
            return jnp.where(cnt >= K, cand, up)

        up = lax.fori_loop(0, 32, bit_body, jnp.int32(0))
        thresh = up ^ _SIGN

        def mask_body(i, _):
            kv = keys[pl.ds(i * L, L)]
            xv = xrow[r, pl.ds(i * L, L)]
            xrow[r, pl.ds(i * L, L)] = jnp.where(kv >= thresh, xv, 0.0)
            return 0

        lax.fori_loop(0, NV, mask_body, 0)

    pltpu.sync_copy(xrow, out_hbm.at[pl.ds(base, ROWS_PER_W)])


@jax.jit
def kernel(x, duty_cycle):
    mesh = plsc.VectorSubcoreMesh(core_axis_name="c", subcore_axis_name="s")
    f = pl.kernel(
        _tile_body,
        out_type=jax.ShapeDtypeStruct((B, N), jnp.float32),
        mesh=mesh,
        scratch_types=[
            pltpu.VMEM((ROWS_PER_W, N), jnp.float32),
            pltpu.VMEM((N,), jnp.float32),
            pltpu.VMEM((N,), jnp.int32),
        ],
        compiler_params=pltpu.CompilerParams(needs_layout_passes=False),
    )
    return f(x, duty_cycle)


# speedup vs baseline: 4.2173x; 2.0955x over previous
import functools

import numpy as np
import jax
import jax.numpy as jnp
from jax import lax
from jax.experimental import pallas as pl
from jax.experimental.pallas import tpu as pltpu
from jax.experimental.pallas import tpu_sc as plsc

N = 8192
B = 64
K = 164
TD = float(K) / float(N)
L = 16
NV = N // L
NC = 2
NS = 16
NW = NC * NS
RPW = B // NW
NB = 512
NG = NB // L

_SIGN = np.int32(-0x80000000)
_MANT = np.int32(0x7FFFFFFF)


def _keyify(pv):
    kb = lax.bitcast_convert_type(pv, jnp.int32)
    return jnp.where(kb < 0, kb ^ _MANT, kb)


def _suffix_incl(v):
    r = lax.rev(v, (0,))
    return lax.rev(plsc.cumsum(r), (0,))


def _tile_body(x_hbm, dc_hbm, out_hbm, xrow, boost, pbuf, bins, hist, cnts,
               offs, cand):
    wid = lax.axis_index("s") * NC + lax.axis_index("c")
    base = wid * RPW
    pltpu.sync_copy(x_hbm.at[pl.ds(base, RPW)], xrow)
    pltpu.sync_copy(dc_hbm, boost)

    iota = lax.iota(jnp.int32, L)
    lane0 = iota == 0

    U = 8

    def boost_body(ib, _):
        for u in range(U):
            s = ib * (U * L) + u * L
            boost[pl.ds(s, L)] = jnp.exp(TD - boost[pl.ds(s, L)])
        return 0

    lax.fori_loop(0, NV // U, boost_body, 0)

    for r in range(RPW):
        def p1_body(ib, carry):
            mnv, mxv = carry
            for u in range(U):
                s = ib * (U * L) + u * L
                pv = xrow[r, pl.ds(s, L)] * boost[pl.ds(s, L)]
                pbuf[pl.ds(s, L)] = pv
                mnv = jnp.minimum(mnv, pv)
                mxv = jnp.maximum(mxv, pv)
            return mnv, mxv

        big = jnp.full((L,), 3.0e38, jnp.float32)
        mnv, mxv = lax.fori_loop(0, NV // U, p1_body, (big, -big))
        mn = jnp.min(mnv)
        mx = jnp.max(mxv)
        dv = jnp.maximum(jnp.full((L,), mx - mn), 1e-30)
        rv = lax.bitcast_convert_type(
            np.int32(0x7EF127EA) - lax.bitcast_convert_type(dv, jnp.int32),
            jnp.float32)
        rv = rv * (2.0 - dv * rv)
        scale = rv * jnp.float32(NB)

        def hz_body(ib, _):
            for u in range(4):
                hist[pl.ds(ib * (4 * L) + u * L, L)] = jnp.zeros((L,), jnp.int32)
            return 0

        lax.fori_loop(0, NG // 4, hz_body, 0)

        def p2_body(ib, _):
            for u in range(U):
                s = ib * (U * L) + u * L
                pv = pbuf[pl.ds(s, L)]
                bv = lax.convert_element_type((pv - mn) * scale, jnp.int32)
                bv = jnp.clip(bv, 0, NB - 1)
                bins[pl.ds(s, L)] = bv
                plsc.addupdate_scatter(hist, [bv], jnp.ones((L,), jnp.int32))
            return 0

        lax.fori_loop(0, NV // U, p2_body, 0)

        def gs_body(g, _):
            v = hist[pl.ds(g * L, L)]
            sg = jnp.sum(v)
            plsc.store_compressed(cnts.at[pl.ds(g, L)], jnp.full((L,), sg),
                                  mask=lane0)
            return 0

        lax.fori_loop(0, NG, gs_body, 0)

        sg1 = cnts[pl.ds(L, L)]
        sg0 = cnts[pl.ds(0, L)]
        suf1 = _suffix_incl(sg1)
        tot1 = suf1[0]
        suf0 = _suffix_incl(sg0) + tot1
        kk = jnp.int32(K)
        ng1 = plsc.all_reduce_population_count(suf1 >= kk)[0]
        ng0 = plsc.all_reduce_population_count(suf0 >= kk)[0]
        gstar = ng0 + ng1 - 1
        gl0 = iota
        gl1 = iota + L
        a_grp = (jnp.sum(jnp.where(gl0 > gstar, sg0, 0))
                 + jnp.sum(jnp.where(gl1 > gstar, sg1, 0)))
        hv = hist[pl.ds(gstar * L, L)]
        sufl = _suffix_incl(hv) + a_grp
        jstar = plsc.all_reduce_population_count(sufl >= kk)[0] - 1
        bstar = gstar * L + jstar
        count_above = a_grp + jnp.sum(jnp.where(iota > jstar, hv, 0))
        need = kk - count_above

        def p3a_body(ib, _):
            for u in range(U):
                i = ib * U + u
                m = bins[pl.ds(i * L, L)] == bstar
                pc = plsc.all_reduce_population_count(m)
                plsc.store_compressed(cnts.at[pl.ds(i, L)], pc, mask=lane0)
            return 0

        lax.fori_loop(0, NV // U, p3a_body, 0)

        def p3b_body(g, carryoff):
            v = cnts[pl.ds(g * L, L)]
            incl = plsc.cumsum(v)
            offs[pl.ds(g * L, L)] = incl - v + carryoff
            return carryoff + incl[L - 1]

        nc = lax.fori_loop(0, NG, p3b_body, jnp.int32(0))

        def p3c_body(ib, _):
            for u in range(4):
                i = ib * 4 + u
                off = offs[pl.ds(i, L)][0]
                m = bins[pl.ds(i * L, L)] == bstar
                kv = _keyify(pbuf[pl.ds(i * L, L)])
                plsc.store_compressed(cand.at[pl.ds(off, L)], kv, mask=m)
            return 0

        lax.fori_loop(0, NV // 4, p3c_body, 0)

        nvr_c = jnp.right_shift(nc + (L - 1), 4)

        def bit_body(bi, up):
            tbit = lax.shift_left(jnp.int32(1), jnp.int32(31) - bi)
            candt = up | tbit
            ts = candt ^ _SIGN

            def cnt_body(i, acc):
                kv = cand[pl.ds(i * L, L)]
                valid = iota < (nc - i * L)
                m = (kv >= ts) & valid
                return acc + jnp.where(m, 1, 0).astype(jnp.int32)

            acc = lax.fori_loop(0, nvr_c, cnt_body, jnp.zeros((L,), jnp.int32))
            cnt = jnp.sum(acc)
            return jnp.where(cnt >= need, candt, up)

        up = lax.fori_loop(0, 32, bit_body, jnp.int32(0))
        tkey = up ^ _SIGN

        def geq_body(i, carry):
            ag, ae = carry
            kv = cand[pl.ds(i * L, L)]
            valid = iota < (nc - i * L)
            ag = ag + jnp.where((kv > tkey) & valid, 1, 0).astype(jnp.int32)
            ae = ae + jnp.where((kv == tkey) & valid, 1, 0).astype(jnp.int32)
            return ag, ae

        zz = jnp.zeros((L,), jnp.int32)
        agv, aev = lax.fori_loop(0, nvr_c, geq_body, (zz, zz))
        cnt_gt = jnp.sum(agv)
        cnt_eq = jnp.sum(aev)
        e_take = need - cnt_gt
        no_tie = cnt_gt + cnt_eq == need

        tkv = jnp.full((L,), tkey)
        tfv = lax.bitcast_convert_type(
            jnp.where(tkv < 0, tkv ^ _MANT, tkv), jnp.float32)

        @pl.when(no_tie)
        def _():
            def p4_body(ib, _):
                for u in range(U):
                    s = ib * (U * L) + u * L
                    pv = pbuf[pl.ds(s, L)]
                    xv = xrow[r, pl.ds(s, L)]
                    xrow[r, pl.ds(s, L)] = jnp.where(pv >= tfv, xv, 0.0)
                return 0

            lax.fori_loop(0, NV // U, p4_body, 0)

        @pl.when(jnp.logical_not(no_tie))
        def _():
            def p4t_body(i, seen):
                s = i * L
                pv = pbuf[pl.ds(s, L)]
                xv = xrow[r, pl.ds(s, L)]
                eq = pv == tfv
                ei = jnp.where(eq, 1, 0).astype(jnp.int32)
                incl = plsc.cumsum(ei)
                take = eq & ((incl - ei + seen) < e_take)
                win = (pv > tfv) | take
                xrow[r, pl.ds(s, L)] = jnp.where(win, xv, 0.0)
                return seen + incl[L - 1]

            lax.fori_loop(0, NV, p4t_body, jnp.int32(0))

    pltpu.sync_copy(xrow, out_hbm.at[pl.ds(base, RPW)])


@jax.jit
def kernel(x, duty_cycle):
    mesh = plsc.VectorSubcoreMesh(core_axis_name="c", subcore_axis_name="s")
    f = pl.kernel(
        _tile_body,
        out_type=jax.ShapeDtypeStruct((B, N), jnp.float32),
        mesh=mesh,
        scratch_types=[
            pltpu.VMEM((RPW, N), jnp.float32),
            pltpu.VMEM((N,), jnp.float32),
            pltpu.VMEM((N,), jnp.float32),
            pltpu.VMEM((N,), jnp.int32),
            pltpu.VMEM((NB,), jnp.int32),
            pltpu.VMEM((NV + L,), jnp.int32),
            pltpu.VMEM((NV + L,), jnp.int32),
            pltpu.VMEM((N + L,), jnp.int32),
        ],
        compiler_params=pltpu.CompilerParams(needs_layout_passes=False),
    )
    return f(x, duty_cycle)


# speedup vs baseline: 7.9766x; 1.8914x over previous
import functools

import numpy as np
import jax
import jax.numpy as jnp
from jax import lax
from jax.experimental import pallas as pl
from jax.experimental.pallas import tpu as pltpu
from jax.experimental.pallas import tpu_sc as plsc

N = 8192
B = 64
K = 164
TD = float(K) / float(N)
L = 16
NV = N // L
NC = 2
NS = 16
NW = NC * NS
RPW = B // NW
NB = 512
NG = NB // L

_SIGN = np.int32(-0x80000000)
_MANT = np.int32(0x7FFFFFFF)


def _keyify(pv):
    kb = lax.bitcast_convert_type(pv, jnp.int32)
    return jnp.where(kb < 0, kb ^ _MANT, kb)


def _suffix_incl(v):
    r = lax.rev(v, (0,))
    return lax.rev(plsc.cumsum(r), (0,))


def _tile_body(x_hbm, dc_hbm, out_hbm, xrow, boost, pbuf, bins, hist, cnts,
               offs, cand):
    wid = lax.axis_index("s") * NC + lax.axis_index("c")
    base = wid * RPW
    pltpu.sync_copy(x_hbm.at[pl.ds(base, RPW)], xrow)
    pltpu.sync_copy(dc_hbm, boost)

    iota = lax.iota(jnp.int32, L)
    lane0 = iota == 0

    U = 8

    def boost_body(ib, _):
        for u in range(U):
            s = ib * (U * L) + u * L
            boost[pl.ds(s, L)] = jnp.exp(TD - boost[pl.ds(s, L)])
        return 0

    lax.fori_loop(0, NV // U, boost_body, 0)

    for r in range(RPW):
        big = jnp.full((L,), 3.0e38, jnp.float32)

        @plsc.parallel_loop(0, NV, unroll=U, carry=(big, -big))
        def p1_loop(i, carry):
            mnv, mxv = carry
            pv = xrow[r, pl.ds(i * L, L)] * boost[pl.ds(i * L, L)]
            pbuf[pl.ds(i * L, L)] = pv
            return jnp.minimum(mnv, pv), jnp.maximum(mxv, pv)

        mnv, mxv = p1_loop
        mn = jnp.min(mnv)
        mx = jnp.max(mxv)
        dv = jnp.maximum(jnp.full((L,), mx - mn), 1e-30)
        rv = lax.bitcast_convert_type(
            np.int32(0x7EF127EA) - lax.bitcast_convert_type(dv, jnp.int32),
            jnp.float32)
        rv = rv * (2.0 - dv * rv)
        scale = rv * jnp.float32(NB)

        def hz_body(ib, _):
            for u in range(4):
                hist[pl.ds(ib * (4 * L) + u * L, L)] = jnp.zeros((L,), jnp.int32)
            return 0

        lax.fori_loop(0, NG // 4, hz_body, 0)

        ones = jnp.ones((L,), jnp.int32)

        @plsc.parallel_loop(0, NV, unroll=U)
        def _(i):
            pv = pbuf[pl.ds(i * L, L)]
            bv = lax.convert_element_type((pv - mn) * scale, jnp.int32)
            bv = jnp.minimum(bv, NB - 1)
            bins[pl.ds(i * L, L)] = bv
            plsc.addupdate_scatter(hist, [bv], ones)

        @plsc.parallel_loop(0, NG, unroll=4)
        def _(g):
            v = hist[pl.ds(g * L, L)]
            sg = jnp.sum(v)
            plsc.store_compressed(cnts.at[pl.ds(g, L)], jnp.full((L,), sg),
                                  mask=lane0)

        sg1 = cnts[pl.ds(L, L)]
        sg0 = cnts[pl.ds(0, L)]
        suf1 = _suffix_incl(sg1)
        tot1 = suf1[0]
        suf0 = _suffix_incl(sg0) + tot1
        kk = jnp.int32(K)
        ng1 = plsc.all_reduce_population_count(suf1 >= kk)[0]
        ng0 = plsc.all_reduce_population_count(suf0 >= kk)[0]
        gstar = ng0 + ng1 - 1
        gl0 = iota
        gl1 = iota + L
        a_grp = (jnp.sum(jnp.where(gl0 > gstar, sg0, 0))
                 + jnp.sum(jnp.where(gl1 > gstar, sg1, 0)))
        hv = hist[pl.ds(gstar * L, L)]
        sufl = _suffix_incl(hv) + a_grp
        jstar = plsc.all_reduce_population_count(sufl >= kk)[0] - 1
        bstar = gstar * L + jstar
        count_above = a_grp + jnp.sum(jnp.where(iota > jstar, hv, 0))
        need = kk - count_above

        @plsc.parallel_loop(0, NV, unroll=U)
        def _(i):
            m = bins[pl.ds(i * L, L)] == bstar
            pc = plsc.all_reduce_population_count(m)
            plsc.store_compressed(cnts.at[pl.ds(i, L)], pc, mask=lane0)

        def p3b_body(g, carryoff):
            v = cnts[pl.ds(g * L, L)]
            incl = plsc.cumsum(v)
            offs[pl.ds(g * L, L)] = incl - v + carryoff
            return carryoff + incl[L - 1]

        nc = lax.fori_loop(0, NG, p3b_body, jnp.int32(0))

        @plsc.parallel_loop(0, NV, unroll=4)
        def _(i):
            off = offs[pl.ds(i, L)][0]
            m = bins[pl.ds(i * L, L)] == bstar
            kv = _keyify(pbuf[pl.ds(i * L, L)])
            plsc.store_compressed(cand.at[pl.ds(off, L)], kv, mask=m)

        nvr_c = jnp.right_shift(nc + (L - 1), 4)

        def bit_body(bi, up):
            tbit = lax.shift_left(jnp.int32(1), jnp.int32(31) - bi)
            candt = up | tbit
            ts = candt ^ _SIGN

            def cnt_body(i, acc):
                kv = cand[pl.ds(i * L, L)]
                valid = iota < (nc - i * L)
                m = (kv >= ts) & valid
                return acc + jnp.where(m, 1, 0).astype(jnp.int32)

            acc = lax.fori_loop(0, nvr_c, cnt_body, jnp.zeros((L,), jnp.int32))
            cnt = jnp.sum(acc)
            return jnp.where(cnt >= need, candt, up)

        up = lax.fori_loop(0, 32, bit_body, jnp.int32(0))
        tkey = up ^ _SIGN

        def geq_body(i, carry):
            ag, ae = carry
            kv = cand[pl.ds(i * L, L)]
            valid = iota < (nc - i * L)
            ag = ag + jnp.where((kv > tkey) & valid, 1, 0).astype(jnp.int32)
            ae = ae + jnp.where((kv == tkey) & valid, 1, 0).astype(jnp.int32)
            return ag, ae

        zz = jnp.zeros((L,), jnp.int32)
        agv, aev = lax.fori_loop(0, nvr_c, geq_body, (zz, zz))
        cnt_gt = jnp.sum(agv)
        cnt_eq = jnp.sum(aev)
        e_take = need - cnt_gt
        no_tie = cnt_gt + cnt_eq == need

        tkv = jnp.full((L,), tkey)
        tfv = lax.bitcast_convert_type(
            jnp.where(tkv < 0, tkv ^ _MANT, tkv), jnp.float32)

        @pl.when(no_tie)
        def _():
            @plsc.parallel_loop(0, NV, unroll=U)
            def _(i):
                pv = pbuf[pl.ds(i * L, L)]
                xv = xrow[r, pl.ds(i * L, L)]
                xrow[r, pl.ds(i * L, L)] = jnp.where(pv >= tfv, xv, 0.0)

        @pl.when(jnp.logical_not(no_tie))
        def _():
            def p4t_body(i, seen):
                s = i * L
                pv = pbuf[pl.ds(s, L)]
                xv = xrow[r, pl.ds(s, L)]
                eq = pv == tfv
                ei = jnp.where(eq, 1, 0).astype(jnp.int32)
                incl = plsc.cumsum(ei)
                take = eq & ((incl - ei + seen) < e_take)
                win = (pv > tfv) | take
                xrow[r, pl.ds(s, L)] = jnp.where(win, xv, 0.0)
                return seen + incl[L - 1]

            lax.fori_loop(0, NV, p4t_body, jnp.int32(0))

    pltpu.sync_copy(xrow, out_hbm.at[pl.ds(base, RPW)])


@jax.jit
def kernel(x, duty_cycle):
    mesh = plsc.VectorSubcoreMesh(core_axis_name="c", subcore_axis_name="s")
    f = pl.kernel(
        _tile_body,
        out_type=jax.ShapeDtypeStruct((B, N), jnp.float32),
        mesh=mesh,
        scratch_types=[
            pltpu.VMEM((RPW, N), jnp.float32),
            pltpu.VMEM((N,), jnp.float32),
            pltpu.VMEM((N,), jnp.float32),
            pltpu.VMEM((N,), jnp.int32),
            pltpu.VMEM((NB,), jnp.int32),
            pltpu.VMEM((NV + L,), jnp.int32),
            pltpu.VMEM((NV + L,), jnp.int32),
            pltpu.VMEM((N + L,), jnp.int32),
        ],
        compiler_params=pltpu.CompilerParams(needs_layout_passes=False),
    )
    return f(x, duty_cycle)
